# baseline (speedup 1.0000x reference)
import functools

import jax
import jax.numpy as jnp
from jax import lax
from jax.experimental import pallas as pl
from jax.experimental.pallas import tpu as pltpu
from jax.experimental.pallas import tpu_sc as plsc

_BATCH = 4
_SEQ = 8192
_HIDDEN = 1024

_NC, _NS, _L = 2, 16, 16
_NW = _NC * _NS
_POS_PER_W = _SEQ // _NW
_R = 8
_NSB = _POS_PER_W // _R
_NSLOT = 3
_PF = _NSLOT - 2


def _sc_body(x_hbm, pos_hbm, out_hbm, pbuf, xbuf, *sems):
    sins = sems[:_NSLOT]
    souts = sems[_NSLOT:]
    wid = lax.axis_index("s") * _NC + lax.axis_index("c")
    pos0 = wid * _POS_PER_W

    def start_in(slot, base):
        pltpu.async_copy(pos_hbm.at[pl.ds(base, _R)], pbuf.at[slot], sins[slot])
        pltpu.async_copy(x_hbm.at[:, pl.ds(base, _R)], xbuf.at[slot], sins[slot])

    def wait_in(slot, base):
        pltpu.make_async_copy(pos_hbm.at[pl.ds(base, _R)], pbuf.at[slot], sins[slot]).wait()
        pltpu.make_async_copy(
            x_hbm.at[:, pl.ds(base, _R)], xbuf.at[slot], sins[slot]
        ).wait()

    def start_out(slot, base):
        pltpu.async_copy(xbuf.at[slot], out_hbm.at[:, pl.ds(base, _R)], souts[slot])

    def wait_out(slot, base):
        pltpu.make_async_copy(
            xbuf.at[slot], out_hbm.at[:, pl.ds(base, _R)], souts[slot]
        ).wait()

    def compute(slot):
        def quarter(j, c):
            for row in range(_R):
                for u in range(16):
                    off = (j * 16 + u) * _L
                    p = pbuf[slot, row, pl.ds(off, _L)]
                    for b in range(_BATCH):
                        plsc.addupdate(xbuf.at[slot, b, row, pl.ds(off, _L)], p)
            return c
        lax.fori_loop(0, _HIDDEN // _L // 16, quarter, 0)

    def process(r, slot, first, last):
        base = pos0 + r * _R
        wait_in(slot, base)
        if not last:
            nxt_slot = (slot + _PF) % _NSLOT
            if first:
                start_in(nxt_slot, base + _PF * _R)
            else:
                @pl.when(r + _PF < _NSB)
                def _():
                    wait_out(nxt_slot, base - (_NSLOT - _PF) * _R)
                    start_in(nxt_slot, base + _PF * _R)
        compute(slot)
        start_out(slot, base)

    for r in range(_PF):
        start_in(r, pos0 + r * _R)

    for r in range(_NSLOT - _PF):
        process(r, r % _NSLOT, first=True, last=False)

    _head = _NSLOT - _PF
    _main = ((_NSB - _head) // _NSLOT) * _NSLOT

    def step(i, carry):
        for s_off in range(_NSLOT):
            r = _head + i * _NSLOT + s_off
            process(r, (_head + s_off) % _NSLOT, first=False, last=False)
        return carry

    lax.fori_loop(0, _main // _NSLOT, step, 0)

    for r in range(_head + _main, _NSB):
        process(r, r % _NSLOT, first=False, last=True)

    for r in range(_NSB - _NSLOT, _NSB):
        wait_out(r % _NSLOT, pos0 + r * _R)


_sc_kernel = functools.partial(
    pl.kernel,
    out_type=jax.ShapeDtypeStruct((_BATCH, _SEQ, _HIDDEN), jnp.float32),
    mesh=plsc.VectorSubcoreMesh(
        core_axis_name="c", subcore_axis_name="s", num_cores=_NC, num_subcores=_NS
    ),
    scratch_types=[
        pltpu.VMEM((_NSLOT, _R, _HIDDEN), jnp.float32),
        pltpu.VMEM((_NSLOT, _BATCH, _R, _HIDDEN), jnp.float32),
    ]
    + [pltpu.SemaphoreType.DMA] * (2 * _NSLOT),
)(_sc_body)


def kernel(x, pos_embedding):
    return _sc_kernel(x, pos_embedding)


# speedup vs baseline: 1.0493x; 1.0493x over previous
import functools

import jax
import jax.numpy as jnp
from jax import lax
from jax.experimental import pallas as pl
from jax.experimental.pallas import tpu as pltpu
from jax.experimental.pallas import tpu_sc as plsc

_BATCH = 4
_SEQ = 8192
_HIDDEN = 1024

_NC, _NS, _L = 2, 16, 16
_NW = _NC * _NS
_POS_PER_W = _SEQ // _NW
_R = 4
_NSB = _POS_PER_W // _R
_NSLOT = 4
_PF = _NSLOT - 2


def _sc_body(x_hbm, pos_hbm, out_hbm, pbuf, xbuf, *sems):
    sins = sems[:_NSLOT]
    souts = sems[_NSLOT:]
    wid = lax.axis_index("s") * _NC + lax.axis_index("c")
    pos0 = wid * _POS_PER_W

    def start_in(slot, base):
        pltpu.async_copy(pos_hbm.at[pl.ds(base, _R)], pbuf.at[slot], sins[slot])
        pltpu.async_copy(x_hbm.at[:, pl.ds(base, _R)], xbuf.at[slot], sins[slot])

    def wait_in(slot, base):
        pltpu.make_async_copy(pos_hbm.at[pl.ds(base, _R)], pbuf.at[slot], sins[slot]).wait()
        pltpu.make_async_copy(
            x_hbm.at[:, pl.ds(base, _R)], xbuf.at[slot], sins[slot]
        ).wait()

    def start_out(slot, base):
        pltpu.async_copy(xbuf.at[slot], out_hbm.at[:, pl.ds(base, _R)], souts[slot])

    def wait_out(slot, base):
        pltpu.make_async_copy(
            xbuf.at[slot], out_hbm.at[:, pl.ds(base, _R)], souts[slot]
        ).wait()

    def compute(slot):
        def quarter(j, c):
            for row in range(_R):
                for u in range(16):
                    off = (j * 16 + u) * _L
                    p = pbuf[slot, row, pl.ds(off, _L)]
                    for b in range(_BATCH):
                        plsc.addupdate(xbuf.at[slot, b, row, pl.ds(off, _L)], p)
            return c
        lax.fori_loop(0, _HIDDEN // _L // 16, quarter, 0)

    def process(r, slot, first, last):
        base = pos0 + r * _R
        wait_in(slot, base)
        if not last:
            nxt_slot = (slot + _PF) % _NSLOT
            if first:
                start_in(nxt_slot, base + _PF * _R)
            else:
                @pl.when(r + _PF < _NSB)
                def _():
                    wait_out(nxt_slot, base - (_NSLOT - _PF) * _R)
                    start_in(nxt_slot, base + _PF * _R)
        compute(slot)
        start_out(slot, base)

    for r in range(_PF):
        start_in(r, pos0 + r * _R)

    for r in range(_NSLOT - _PF):
        process(r, r % _NSLOT, first=True, last=False)

    _head = _NSLOT - _PF
    _main = ((_NSB - _head) // _NSLOT) * _NSLOT

    def step(i, carry):
        for s_off in range(_NSLOT):
            r = _head + i * _NSLOT + s_off
            process(r, (_head + s_off) % _NSLOT, first=False, last=False)
        return carry

    lax.fori_loop(0, _main // _NSLOT, step, 0)

    for r in range(_head + _main, _NSB):
        process(r, r % _NSLOT, first=False, last=True)

    for r in range(_NSB - _NSLOT, _NSB):
        wait_out(r % _NSLOT, pos0 + r * _R)


_sc_kernel = functools.partial(
    pl.kernel,
    out_type=jax.ShapeDtypeStruct((_BATCH, _SEQ, _HIDDEN), jnp.float32),
    mesh=plsc.VectorSubcoreMesh(
        core_axis_name="c", subcore_axis_name="s", num_cores=_NC, num_subcores=_NS
    ),
    scratch_types=[
        pltpu.VMEM((_NSLOT, _R, _HIDDEN), jnp.float32),
        pltpu.VMEM((_NSLOT, _BATCH, _R, _HIDDEN), jnp.float32),
    ]
    + [pltpu.SemaphoreType.DMA] * (2 * _NSLOT),
)(_sc_body)


def kernel(x, pos_embedding):
    return _sc_kernel(x, pos_embedding)
